# baseline (speedup 1.0000x reference)
import functools

import jax
import jax.numpy as jnp
from jax.experimental import pallas as pl
from jax.experimental.pallas import tpu as pltpu

BLK = 256


def _vq_block_kernel(z_ref, wt_ref, w_ref, q_ref, idx_ref, cnt_ref, loss_ref):
    blk, k = z_ref.shape[0], wt_ref.shape[1]
    z = z_ref[...]
    wt = wt_ref[...]
    z2 = jnp.sum(z * z, axis=1, keepdims=True)
    w2 = jnp.sum(wt * wt, axis=0, keepdims=True)
    scores = jnp.dot(z, wt, preferred_element_type=jnp.float32)
    dist = (z2 + w2) - 2.0 * scores
    m = jnp.min(dist, axis=1, keepdims=True)
    lanes = jax.lax.broadcasted_iota(jnp.int32, (blk, k), 1)
    idx = jnp.min(jnp.where(dist == m, lanes, k), axis=1, keepdims=True)
    onehot = (lanes == idx).astype(jnp.float32)
    q_ref[...] = jnp.dot(onehot, w_ref[...], preferred_element_type=jnp.float32)
    idx_ref[...] = idx
    cnt_ref[...] = jnp.sum(onehot, axis=0, keepdims=True)[None]
    loss_ref[...] = jnp.full((1, 1, 128), jnp.sum(m), dtype=jnp.float32)


def _finalize_kernel(cnt_ref, loss_ref, loss_out, perp_out, *, n_tok, dim):
    counts = jnp.sum(cnt_ref[...], axis=0)
    probs = counts * (1.0 / n_tok)
    ent = -jnp.sum(probs * jnp.log(probs + 1e-10))
    perp_out[...] = jnp.full((1, 1), jnp.exp(ent), dtype=jnp.float32)
    total = jnp.sum(loss_ref[...]) * (1.0 / 128.0)
    loss_out[...] = jnp.full((1, 1), 0.25 * total / (n_tok * dim), dtype=jnp.float32)


def _vq_forward(z_e, w, blk, interpret=False):
    n_tok, dim = z_e.shape
    n_emb = w.shape[0]
    nb = n_tok // blk
    q, idx, cnt_p, loss_p = pl.pallas_call(
        _vq_block_kernel,
        grid=(nb,),
        in_specs=[
            pl.BlockSpec((blk, dim), lambda i: (i, 0)),
            pl.BlockSpec((dim, n_emb), lambda i: (0, 0)),
            pl.BlockSpec((n_emb, dim), lambda i: (0, 0)),
        ],
        out_specs=[
            pl.BlockSpec((blk, dim), lambda i: (i, 0)),
            pl.BlockSpec((blk, 1), lambda i: (i, 0)),
            pl.BlockSpec((1, 1, n_emb), lambda i: (i, 0, 0)),
            pl.BlockSpec((1, 1, 128), lambda i: (i, 0, 0)),
        ],
        out_shape=[
            jax.ShapeDtypeStruct((n_tok, dim), jnp.float32),
            jax.ShapeDtypeStruct((n_tok, 1), jnp.int32),
            jax.ShapeDtypeStruct((nb, 1, n_emb), jnp.float32),
            jax.ShapeDtypeStruct((nb, 1, 128), jnp.float32),
        ],
        compiler_params=pltpu.CompilerParams(
            dimension_semantics=("arbitrary",),
        ),
        interpret=interpret,
    )(z_e, w.T, w)
    loss, perp = pl.pallas_call(
        functools.partial(_finalize_kernel, n_tok=n_tok, dim=dim),
        out_shape=[
            jax.ShapeDtypeStruct((1, 1), jnp.float32),
            jax.ShapeDtypeStruct((1, 1), jnp.float32),
        ],
        interpret=interpret,
    )(cnt_p, loss_p)
    return loss.reshape(()), q, perp.reshape(()), idx


def kernel(z_e, embedding_weight):
    loss, q, perp, _ = _vq_forward(z_e, embedding_weight, BLK)
    return (loss, q, perp)
